# Initial kernel scaffold; baseline (speedup 1.0000x reference)
import jax
import jax.numpy as jnp
from jax.experimental import pallas as pl


def kernel(input_ids, labels, emb_table):
    raise NotImplementedError("write your pallas kernel here")


# speedup vs baseline: 1.1956x; 1.1956x over previous
import functools
import math

import jax
import jax.numpy as jnp
from jax import lax
from jax.experimental import pallas as pl
from jax.experimental.pallas import tpu as pltpu
from jax.experimental.pallas import tpu_sc as plsc

_ROT_DIM = 64

_NC = 2
_NS = 16
_NW = _NC * _NS

_C = 8
_NBUF = 2


def _emb_gather_sc(emb_table, idx_flat):
    n_rows = idx_flat.shape[0]
    d = emb_table.shape[1]
    rows_per_w = n_rows // _NW
    nchunk = rows_per_w // _C
    ngroup = nchunk // _NBUF
    assert rows_per_w * _NW == n_rows
    assert _C * nchunk == rows_per_w and _NBUF * ngroup == nchunk

    mesh = plsc.VectorSubcoreMesh(core_axis_name="c", subcore_axis_name="s")

    @functools.partial(
        pl.kernel,
        mesh=mesh,
        out_type=jax.ShapeDtypeStruct((n_rows, d), jnp.float32),
        scratch_types=(
            [pltpu.VMEM((rows_per_w,), jnp.int32),
             pltpu.VMEM((_NBUF, _C, d), jnp.float32)]
            + [pltpu.SemaphoreType.DMA] * (2 * _NBUF)
        ),
    )
    def k(table_hbm, idx_hbm, out_hbm, idx_v, rows_v, *sems):
        gsem = sems[:_NBUF]
        ssem = sems[_NBUF:]
        wid = lax.axis_index("s") * _NC + lax.axis_index("c")
        base = wid * rows_per_w
        pltpu.sync_copy(idx_hbm.at[pl.ds(base, rows_per_w)], idx_v)

        def start_gather(g, b):
            pltpu.async_copy(
                table_hbm.at[idx_v.at[pl.ds(g * _C, _C)]],
                rows_v.at[b], gsem[b])

        def wait_gather(b):
            pltpu.make_async_copy(
                table_hbm.at[idx_v.at[pl.ds(0, _C)]],
                rows_v.at[b], gsem[b]).wait()

        def start_scatter(g, b):
            pltpu.async_copy(
                rows_v.at[b],
                out_hbm.at[pl.ds(base + g * _C, _C)], ssem[b])

        def wait_scatter(b):
            pltpu.make_async_copy(
                rows_v.at[b],
                out_hbm.at[pl.ds(0, _C)], ssem[b]).wait()

        for b in range(_NBUF):
            start_gather(b, b)

        def group(gi, carry):
            gprev = (gi - 1) * _NBUF
            gcur = gi * _NBUF
            for b in range(_NBUF):
                wait_gather(b)
                start_scatter(gprev + b, b)
            for b in range(_NBUF):
                wait_scatter(b)
                start_gather(gcur + b, b)
            return carry

        lax.fori_loop(1, ngroup, group, 0)

        last = (ngroup - 1) * _NBUF
        for b in range(_NBUF):
            wait_gather(b)
            start_scatter(last + b, b)
        for b in range(_NBUF):
            wait_scatter(b)

    return k(emb_table, idx_flat)


def _rotary_tc(seq_len):
    def body(out_ref):
        s = lax.broadcasted_iota(jnp.int32, (seq_len, _ROT_DIM), 0).astype(jnp.float32)
        j = lax.broadcasted_iota(jnp.int32, (seq_len, _ROT_DIM), 1)
        i = (j // 2).astype(jnp.float32)
        inv_freq = jnp.exp(i * (-math.log(10000.0) / (_ROT_DIM // 2)))
        ang = s * inv_freq
        out_ref[...] = jnp.where(j % 2 == 0, jnp.cos(ang), jnp.sin(ang))

    return pl.pallas_call(
        body,
        out_shape=jax.ShapeDtypeStruct((seq_len, _ROT_DIM), jnp.float32),
    )()


def kernel(input_ids, labels, emb_table):
    b, s = input_ids.shape
    hidden = emb_table.shape[1]
    idx_flat = jnp.transpose(input_ids).reshape(-1)
    flat = _emb_gather_sc(emb_table, idx_flat)
    hidden_states = flat.reshape(s, b, hidden)
    rotary = _rotary_tc(s).reshape(s, 1, _ROT_DIM // 2, 2)
    return (hidden_states, rotary, labels)
